# Initial kernel scaffold; baseline (speedup 1.0000x reference)
import jax
import jax.numpy as jnp
from jax.experimental import pallas as pl


def kernel(feature_maps, cell_masks, cell_counts):
    raise NotImplementedError("write your pallas kernel here")


# speedup vs baseline: 1.8092x; 1.8092x over previous
import jax
import jax.numpy as jnp
from jax.experimental import pallas as pl
from jax.experimental.pallas import tpu as pltpu


def _roipool_body(batch_idx_ref, fm_ref, mask_ref, out_ref):
    fm = fm_ref[0]
    m = mask_ref[0]
    neg = jnp.finfo(fm.dtype).min
    masked = jnp.where(m[None, :, :] != 0, fm, neg)
    out_ref[0, 0, :] = jnp.max(masked, axis=(1, 2))


def kernel(feature_maps, cell_masks, cell_counts):
    B, C, H, W = feature_maps.shape
    n_cells = cell_masks.shape[0]
    HW = H * W
    fm = feature_maps.reshape(B, C, HW // 128, 128)
    masks = cell_masks.reshape(n_cells, HW // 128, 128).astype(jnp.int8)

    ends = jnp.cumsum(cell_counts)
    batch_idx = jnp.searchsorted(ends, jnp.arange(n_cells), side="right")
    batch_idx = batch_idx.astype(jnp.int32)

    grid_spec = pltpu.PrefetchScalarGridSpec(
        num_scalar_prefetch=1,
        grid=(n_cells,),
        in_specs=[
            pl.BlockSpec(
                (1, C, HW // 128, 128),
                lambda i, bidx: (bidx[i], 0, 0, 0),
            ),
            pl.BlockSpec(
                (1, HW // 128, 128),
                lambda i, bidx: (i, 0, 0),
            ),
        ],
        out_specs=pl.BlockSpec((1, 1, C), lambda i, bidx: (i, 0, 0)),
    )

    out = pl.pallas_call(
        _roipool_body,
        grid_spec=grid_spec,
        out_shape=jax.ShapeDtypeStruct((n_cells, 1, C), feature_maps.dtype),
    )(batch_idx, fm, masks)
    return out.reshape(n_cells, C)
